# baseline (speedup 1.0000x reference)
import functools

import numpy as np
import jax
import jax.numpy as jnp
from jax import lax
from jax.experimental import pallas as pl
from jax.experimental.pallas import tpu as pltpu
from jax.experimental.pallas import tpu_sc as plsc

EMBED = 128
LANES = 16
NC, NS = 2, 16
NW = NC * NS
NBUF = 8


def _positional_encoding_np(length: int, depth: int) -> np.ndarray:
    half = depth // 2
    positions = np.arange(length, dtype=np.float32)[:, None]
    depths = (np.arange(half, dtype=np.float32)[None, :] / float(half))
    angle_rates = (1.0 / (10000.0 ** depths)).astype(np.float32)
    angle_rads = positions * angle_rates
    return np.concatenate(
        [np.sin(angle_rads), np.cos(angle_rads)], axis=-1
    ).astype(np.float32)


@functools.cache
def _make_sc_kernel(batch: int, seq: int, chunk: int):
    b_total = batch * seq
    bpw = b_total // NW
    nchunks = bpw // chunk
    spw = batch // NW
    phases = seq // chunk
    assert nchunks % NBUF == 0 and seq % chunk == 0
    mesh = plsc.VectorSubcoreMesh(
        core_axis_name="c", subcore_axis_name="s",
        num_cores=NC, num_subcores=NS,
    )

    def out_block(out_hbm, wid, ci):
        return out_hbm.at[wid * spw + ci // phases,
                          pl.ds((ci % phases) * chunk, chunk)]

    @functools.partial(
        pl.kernel,
        out_type=jax.ShapeDtypeStruct((batch, seq, EMBED), jnp.float32),
        mesh=mesh,
        scratch_types=[
            pltpu.VMEM((nchunks, chunk), jnp.int32),
            pltpu.VMEM((NBUF, chunk, EMBED), jnp.float32),
            pltpu.VMEM((seq, EMBED), jnp.float32),
        ] + [pltpu.SemaphoreType.DMA] * (2 * NBUF),
    )
    def sc_kernel(idx_hbm, table_hbm, pe_hbm, out_hbm, idx_v, rows_v, pe_v,
                  *sems):
        gsem = sems[:NBUF]
        ssem = sems[NBUF:]
        wid = lax.axis_index("s") * NC + lax.axis_index("c")
        pltpu.sync_copy(idx_hbm.at[wid], idx_v)
        pltpu.sync_copy(pe_hbm, pe_v)

        for b in range(NBUF):
            pltpu.async_copy(table_hbm.at[idx_v.at[b]], rows_v.at[b], gsem[b])

        def add_pe(b, ci):
            pe_off = (ci % phases) * chunk

            def row_body(r, _):
                pr = pe_off + r
                for j in range(EMBED // LANES):
                    sl = pl.ds(j * LANES, LANES)
                    rows_v[b, r, sl] = rows_v[b, r, sl] + pe_v[pr, sl]
                return 0

            lax.fori_loop(0, chunk, row_body, 0, unroll=2)

        @pl.loop(0, nchunks, step=NBUF)
        def _(ci0):
            for b in range(NBUF):
                ci = ci0 + b
                pltpu.make_async_copy(
                    table_hbm.at[idx_v.at[ci]], rows_v.at[b], gsem[b]).wait()
                add_pe(b, ci)
                pltpu.async_copy(rows_v.at[b], out_block(out_hbm, wid, ci),
                                 ssem[b])
                cg = ci + NBUF - 1
                bb = (b - 1) % NBUF

                @pl.when(jnp.logical_and(cg >= NBUF, cg < nchunks))
                def _():
                    pltpu.make_async_copy(
                        rows_v.at[bb], out_block(out_hbm, wid, ci - 1),
                        ssem[bb]).wait()
                    pltpu.async_copy(
                        table_hbm.at[idx_v.at[cg]], rows_v.at[bb], gsem[bb])

        for b in range(NBUF):
            ci = nchunks - NBUF + b
            pltpu.make_async_copy(
                rows_v.at[b], out_block(out_hbm, wid, ci), ssem[b]).wait()

    return sc_kernel


def kernel(x, table):
    batch, seq = x.shape
    b_total = batch * seq
    chunk = 40
    idx = x.reshape(NW, (b_total // NW) // chunk, chunk).astype(jnp.int32)
    pe = jnp.asarray(_positional_encoding_np(seq, EMBED))
    sc = _make_sc_kernel(batch, seq, chunk)
    return sc(idx, table, pe)


# speedup vs baseline: 2.3703x; 2.3703x over previous
import functools

import numpy as np
import jax
import jax.numpy as jnp
from jax import lax
from jax.experimental import pallas as pl
from jax.experimental.pallas import tpu as pltpu
from jax.experimental.pallas import tpu_sc as plsc

EMBED = 128
LANES = 16
NC, NS = 2, 16
NW = NC * NS
NSB = 3
HALF = 100


def _positional_encoding_np(length: int, depth: int) -> np.ndarray:
    half = depth // 2
    positions = np.arange(length, dtype=np.float32)[:, None]
    depths = (np.arange(half, dtype=np.float32)[None, :] / float(half))
    angle_rates = (1.0 / (10000.0 ** depths)).astype(np.float32)
    angle_rads = positions * angle_rates
    return np.concatenate(
        [np.sin(angle_rads), np.cos(angle_rads)], axis=-1
    ).astype(np.float32)


@functools.cache
def _make_sc_kernel(batch: int, seq: int):
    spw = batch // NW
    halves = seq // HALF
    main = spw - spw % NSB
    mesh = plsc.VectorSubcoreMesh(
        core_axis_name="c", subcore_axis_name="s",
        num_cores=NC, num_subcores=NS,
    )

    @functools.partial(
        pl.kernel,
        out_type=jax.ShapeDtypeStruct((batch, seq, EMBED), jnp.float32),
        mesh=mesh,
        scratch_types=[
            pltpu.VMEM((spw * halves, HALF), jnp.int32),
            pltpu.VMEM((NSB, seq, EMBED), jnp.float32),
            pltpu.VMEM((seq, EMBED), jnp.float32),
        ] + [pltpu.SemaphoreType.DMA] * (2 * NSB),
    )
    def sc_kernel(idx_hbm, table_hbm, pe_hbm, out_hbm, idx_v, rows_v, pe_v,
                  *sems):
        gsem = sems[:NSB]
        ssem = sems[NSB:]
        wid = lax.axis_index("s") * NC + lax.axis_index("c")
        pltpu.sync_copy(idx_hbm.at[wid], idx_v)
        pltpu.sync_copy(pe_hbm, pe_v)

        def gather(si, b):
            for h in range(halves):
                pltpu.async_copy(
                    table_hbm.at[idx_v.at[si * halves + h]],
                    rows_v.at[b, pl.ds(h * HALF, HALF)], gsem[b])

        def wait_gather(si, b):
            for h in range(halves):
                pltpu.make_async_copy(
                    table_hbm.at[idx_v.at[si * halves + h]],
                    rows_v.at[b, pl.ds(h * HALF, HALF)], gsem[b]).wait()

        def add_pe(b):
            def row_body(r, _):
                for j in range(EMBED // LANES):
                    sl = pl.ds(j * LANES, LANES)
                    plsc.addupdate(rows_v.at[b, r, sl], pe_v[r, sl])
                return 0

            lax.fori_loop(0, seq, row_body, 0, unroll=2)

        def store(si, b):
            pltpu.async_copy(rows_v.at[b], out_hbm.at[wid * spw + si], ssem[b])

        def wait_store(si, b):
            pltpu.make_async_copy(
                rows_v.at[b], out_hbm.at[wid * spw + si], ssem[b]).wait()

        def step(si, b):
            wait_gather(si, b)
            cg = si + NSB - 1
            bb = (b + NSB - 1) % NSB

            @pl.when(jnp.logical_and(cg >= NSB, cg < spw))
            def _():
                wait_store(si - 1, bb)
                gather(cg, bb)

            add_pe(b)
            store(si, b)

        for b in range(NSB):
            gather(b, b)

        @pl.loop(0, main, step=NSB)
        def _(si0):
            for b in range(NSB):
                step(si0 + b, b)

        for si in range(main, spw):
            step(si, si % NSB)

        for si in range(spw - NSB, spw):
            wait_store(si, si % NSB)

    return sc_kernel


def kernel(x, table):
    batch, seq = x.shape
    idx = x.reshape(NW, (batch // NW) * (seq // HALF), HALF).astype(jnp.int32)
    pe = jnp.asarray(_positional_encoding_np(seq, EMBED))
    sc = _make_sc_kernel(batch, seq)
    return sc(idx, table, pe)
